# baseline (speedup 1.0000x reference)
import functools

import jax
import jax.numpy as jnp
from jax import lax
from jax.experimental import pallas as pl
from jax.experimental.pallas import tpu as pltpu
from jax.experimental.pallas import tpu_sc as plsc

_N_NODES = 10000
_N_EDGES = 320000
_EMB = 128
_NUM_RADIAL = 16
_VOCAB = 100

_NC = 2
_NS = 16
_L = 16
_NW = _NC * _NS
_PER_W = _N_EDGES // _NW

_E_BLK = 20000
_NB = _N_EDGES // _E_BLK


def _sc_compose_indices(Z, idnb_i, idnb_j):
    mesh = plsc.VectorSubcoreMesh(
        core_axis_name="c", subcore_axis_name="s",
        num_cores=_NC, num_subcores=_NS)

    @functools.partial(
        pl.kernel,
        out_type=(jax.ShapeDtypeStruct((_N_EDGES,), jnp.int32),
                  jax.ShapeDtypeStruct((_N_EDGES,), jnp.int32)),
        mesh=mesh,
        scratch_types=[
            pltpu.VMEM((_N_NODES,), jnp.int32),
            pltpu.VMEM((_PER_W,), jnp.int32),
            pltpu.VMEM((_PER_W,), jnp.int32),
            pltpu.VMEM((_PER_W,), jnp.int32),
            pltpu.VMEM((_PER_W,), jnp.int32),
            pltpu.SemaphoreType.DMA,
            pltpu.SemaphoreType.DMA,
            pltpu.SemaphoreType.DMA,
        ],
        compiler_params=pltpu.CompilerParams(
            needs_layout_passes=False, skip_device_barrier=True),
    )
    def k(z_hbm, ii_hbm, jj_hbm, zi_hbm, zj_hbm,
          z_v, ii_v, jj_v, oi_v, oj_v, sem_z, sem_i, sem_j):
        wid = lax.axis_index("s") * _NC + lax.axis_index("c")
        base = wid * _PER_W
        cp_z = pltpu.async_copy(z_hbm, z_v, sem_z)
        cp_i = pltpu.async_copy(ii_hbm.at[pl.ds(base, _PER_W)], ii_v, sem_i)
        cp_j = pltpu.async_copy(jj_hbm.at[pl.ds(base, _PER_W)], jj_v, sem_j)
        cp_z.wait()
        cp_i.wait()

        @plsc.parallel_loop(0, _PER_W, _L, unroll=8)
        def _(i):
            oi_v[pl.ds(i, _L)] = plsc.load_gather(z_v, [ii_v[pl.ds(i, _L)]])

        cp_oi = pltpu.async_copy(oi_v, zi_hbm.at[pl.ds(base, _PER_W)], sem_i)
        cp_j.wait()

        @plsc.parallel_loop(0, _PER_W, _L, unroll=8)
        def _(i):
            oj_v[pl.ds(i, _L)] = plsc.load_gather(z_v, [jj_v[pl.ds(i, _L)]])

        cp_oj = pltpu.async_copy(oj_v, zj_hbm.at[pl.ds(base, _PER_W)], sem_j)
        cp_oi.wait()
        cp_oj.wait()

    return k(Z, idnb_i, idnb_j)


def _tc_body(zi_ref, zj_ref, rbf_ref, emb_ref, wr_ref, br_ref, wd_ref, bd_ref,
             out_ref, t1_s, t2_s, wc_s, bc_s):
    @pl.when(pl.program_id(0) == 0)
    def _():
        w3 = wd_ref[2 * _EMB:3 * _EMB, :]
        t1_s[...] = jnp.dot(emb_ref[...], wd_ref[0:_EMB, :],
                            preferred_element_type=jnp.float32).astype(jnp.bfloat16)
        t2_s[...] = jnp.dot(emb_ref[...], wd_ref[_EMB:2 * _EMB, :],
                            preferred_element_type=jnp.float32).astype(jnp.bfloat16)
        wc_s[...] = jnp.dot(wr_ref[...], w3, preferred_element_type=jnp.float32)
        bc_s[...] = jnp.dot(br_ref[...], w3,
                            preferred_element_type=jnp.float32) + bd_ref[...]

    zi = zi_ref[0]
    zj = zj_ref[0]
    row = lax.broadcasted_iota(jnp.int32, (_EMB, _E_BLK), 0)
    oht_i = (row == zi).astype(jnp.bfloat16)
    oht_j = (row == zj).astype(jnp.bfloat16)
    dn = (((0,), (0,)), ((), ()))
    acc = lax.dot_general(oht_i, t1_s[...], dn,
                          preferred_element_type=jnp.float32)
    acc = acc + lax.dot_general(oht_j, t2_s[...], dn,
                                preferred_element_type=jnp.float32)
    acc = acc + jnp.dot(rbf_ref[...], wc_s[...],
                        preferred_element_type=jnp.float32)
    out_ref[...] = acc + bc_s[...]


def _tc_fused(zi3, zj3, rbf, emb_pad, W_rbf, b_rbf2, W_dense, b_dense2):
    return pl.pallas_call(
        _tc_body,
        grid=(_NB,),
        in_specs=[
            pl.BlockSpec((1, 1, _E_BLK), lambda i: (i, 0, 0)),
            pl.BlockSpec((1, 1, _E_BLK), lambda i: (i, 0, 0)),
            pl.BlockSpec((_E_BLK, _NUM_RADIAL), lambda i: (i, 0)),
            pl.BlockSpec((_EMB, _EMB), lambda i: (0, 0)),
            pl.BlockSpec((_NUM_RADIAL, _EMB), lambda i: (0, 0)),
            pl.BlockSpec((1, _EMB), lambda i: (0, 0)),
            pl.BlockSpec((3 * _EMB, _EMB), lambda i: (0, 0)),
            pl.BlockSpec((1, _EMB), lambda i: (0, 0)),
        ],
        out_specs=pl.BlockSpec((_E_BLK, _EMB), lambda i: (i, 0)),
        out_shape=jax.ShapeDtypeStruct((_N_EDGES, _EMB), jnp.float32),
        scratch_shapes=[
            pltpu.VMEM((_EMB, _EMB), jnp.bfloat16),
            pltpu.VMEM((_EMB, _EMB), jnp.bfloat16),
            pltpu.VMEM((_NUM_RADIAL, _EMB), jnp.float32),
            pltpu.VMEM((1, _EMB), jnp.float32),
        ],
        compiler_params=pltpu.CompilerParams(
            dimension_semantics=("arbitrary",)),
    )(zi3, zj3, rbf, emb_pad, W_rbf, b_rbf2, W_dense, b_dense2)


def kernel(Z, rbf, idnb_i, idnb_j, emb_table, W_rbf, b_rbf, W_dense, b_dense):
    Z = Z.astype(jnp.int32)
    idnb_i = idnb_i.astype(jnp.int32)
    idnb_j = idnb_j.astype(jnp.int32)

    zi, zj = _sc_compose_indices(Z, idnb_i, idnb_j)
    zi3 = zi.reshape(_NB, 1, _E_BLK)
    zj3 = zj.reshape(_NB, 1, _E_BLK)

    emb_pad = jnp.zeros((_EMB, _EMB), jnp.float32).at[:_VOCAB].set(emb_table)
    out = _tc_fused(zi3, zj3, rbf, emb_pad, W_rbf,
                    b_rbf.reshape(1, _EMB), W_dense, b_dense.reshape(1, _EMB))
    return out


# speedup vs baseline: 1.0123x; 1.0123x over previous
import functools

import jax
import jax.numpy as jnp
from jax import lax
from jax.experimental import pallas as pl
from jax.experimental.pallas import tpu as pltpu
from jax.experimental.pallas import tpu_sc as plsc

_N_NODES = 10000
_N_EDGES = 320000
_EMB = 128
_NUM_RADIAL = 16
_VOCAB = 100

_NC = 1
_NS = 16
_L = 16
_NW = _NC * _NS
_PER_W = _N_EDGES // _NW

_E_BLK = 20000
_NB = _N_EDGES // _E_BLK


def _sc_compose_indices(Z, idnb_i, idnb_j):
    mesh = plsc.VectorSubcoreMesh(
        core_axis_name="c", subcore_axis_name="s",
        num_cores=_NC, num_subcores=_NS)

    @functools.partial(
        pl.kernel,
        out_type=(jax.ShapeDtypeStruct((_N_EDGES,), jnp.int32),
                  jax.ShapeDtypeStruct((_N_EDGES,), jnp.int32)),
        mesh=mesh,
        scratch_types=[
            pltpu.VMEM((_N_NODES,), jnp.int32),
            pltpu.VMEM((_PER_W,), jnp.int32),
            pltpu.VMEM((_PER_W,), jnp.int32),
            pltpu.VMEM((_PER_W,), jnp.int32),
            pltpu.VMEM((_PER_W,), jnp.int32),
            pltpu.SemaphoreType.DMA,
            pltpu.SemaphoreType.DMA,
            pltpu.SemaphoreType.DMA,
        ],
        compiler_params=pltpu.CompilerParams(needs_layout_passes=False),
    )
    def k(z_hbm, ii_hbm, jj_hbm, zi_hbm, zj_hbm,
          z_v, ii_v, jj_v, oi_v, oj_v, sem_z, sem_i, sem_j):
        wid = lax.axis_index("s") * _NC + lax.axis_index("c")
        base = wid * _PER_W
        cp_z = pltpu.async_copy(z_hbm, z_v, sem_z)
        cp_i = pltpu.async_copy(ii_hbm.at[pl.ds(base, _PER_W)], ii_v, sem_i)
        cp_j = pltpu.async_copy(jj_hbm.at[pl.ds(base, _PER_W)], jj_v, sem_j)
        cp_z.wait()
        cp_i.wait()

        @plsc.parallel_loop(0, _PER_W, _L, unroll=8)
        def _(i):
            oi_v[pl.ds(i, _L)] = plsc.load_gather(z_v, [ii_v[pl.ds(i, _L)]])

        cp_oi = pltpu.async_copy(oi_v, zi_hbm.at[pl.ds(base, _PER_W)], sem_i)
        cp_j.wait()

        @plsc.parallel_loop(0, _PER_W, _L, unroll=8)
        def _(i):
            oj_v[pl.ds(i, _L)] = plsc.load_gather(z_v, [jj_v[pl.ds(i, _L)]])

        cp_oj = pltpu.async_copy(oj_v, zj_hbm.at[pl.ds(base, _PER_W)], sem_j)
        cp_oi.wait()
        cp_oj.wait()

    return k(Z, idnb_i, idnb_j)


def _tc_body(zi_ref, zj_ref, rbf_ref, emb_ref, wr_ref, br_ref, wd_ref, bd_ref,
             out_ref, t1_s, t2_s, wc_s, bc_s):
    @pl.when(pl.program_id(0) == 0)
    def _():
        w3 = wd_ref[2 * _EMB:3 * _EMB, :]
        t1_s[...] = jnp.dot(emb_ref[...], wd_ref[0:_EMB, :],
                            preferred_element_type=jnp.float32).astype(jnp.bfloat16)
        t2_s[...] = jnp.dot(emb_ref[...], wd_ref[_EMB:2 * _EMB, :],
                            preferred_element_type=jnp.float32).astype(jnp.bfloat16)
        wc_s[...] = jnp.dot(wr_ref[...], w3, preferred_element_type=jnp.float32)
        bc_s[...] = jnp.dot(br_ref[...], w3,
                            preferred_element_type=jnp.float32) + bd_ref[...]

    zi = zi_ref[0]
    zj = zj_ref[0]
    row = lax.broadcasted_iota(jnp.int32, (_EMB, _E_BLK), 0)
    oht_i = (row == zi).astype(jnp.bfloat16)
    oht_j = (row == zj).astype(jnp.bfloat16)
    dn = (((0,), (0,)), ((), ()))
    acc = lax.dot_general(oht_i, t1_s[...], dn,
                          preferred_element_type=jnp.float32)
    acc = acc + lax.dot_general(oht_j, t2_s[...], dn,
                                preferred_element_type=jnp.float32)
    acc = acc + jnp.dot(rbf_ref[...], wc_s[...],
                        preferred_element_type=jnp.float32)
    out_ref[...] = acc + bc_s[...]


def _tc_fused(zi3, zj3, rbf, emb_pad, W_rbf, b_rbf2, W_dense, b_dense2):
    return pl.pallas_call(
        _tc_body,
        grid=(_NB,),
        in_specs=[
            pl.BlockSpec((1, 1, _E_BLK), lambda i: (i, 0, 0)),
            pl.BlockSpec((1, 1, _E_BLK), lambda i: (i, 0, 0)),
            pl.BlockSpec((_E_BLK, _NUM_RADIAL), lambda i: (i, 0)),
            pl.BlockSpec((_EMB, _EMB), lambda i: (0, 0)),
            pl.BlockSpec((_NUM_RADIAL, _EMB), lambda i: (0, 0)),
            pl.BlockSpec((1, _EMB), lambda i: (0, 0)),
            pl.BlockSpec((3 * _EMB, _EMB), lambda i: (0, 0)),
            pl.BlockSpec((1, _EMB), lambda i: (0, 0)),
        ],
        out_specs=pl.BlockSpec((_E_BLK, _EMB), lambda i: (i, 0)),
        out_shape=jax.ShapeDtypeStruct((_N_EDGES, _EMB), jnp.float32),
        scratch_shapes=[
            pltpu.VMEM((_EMB, _EMB), jnp.bfloat16),
            pltpu.VMEM((_EMB, _EMB), jnp.bfloat16),
            pltpu.VMEM((_NUM_RADIAL, _EMB), jnp.float32),
            pltpu.VMEM((1, _EMB), jnp.float32),
        ],
        compiler_params=pltpu.CompilerParams(
            dimension_semantics=("arbitrary",)),
    )(zi3, zj3, rbf, emb_pad, W_rbf, b_rbf2, W_dense, b_dense2)


def kernel(Z, rbf, idnb_i, idnb_j, emb_table, W_rbf, b_rbf, W_dense, b_dense):
    Z = Z.astype(jnp.int32)
    idnb_i = idnb_i.astype(jnp.int32)
    idnb_j = idnb_j.astype(jnp.int32)

    zi, zj = _sc_compose_indices(Z, idnb_i, idnb_j)
    zi3 = zi.reshape(_NB, 1, _E_BLK)
    zj3 = zj.reshape(_NB, 1, _E_BLK)

    emb_pad = jnp.zeros((_EMB, _EMB), jnp.float32).at[:_VOCAB].set(emb_table)
    out = _tc_fused(zi3, zj3, rbf, emb_pad, W_rbf,
                    b_rbf.reshape(1, _EMB), W_dense, b_dense.reshape(1, _EMB))
    return out
